# Initial kernel scaffold; baseline (speedup 1.0000x reference)
import jax
import jax.numpy as jnp
from jax.experimental import pallas as pl


def kernel(input, target):
    raise NotImplementedError("write your pallas kernel here")


# speedup vs baseline: 1.8071x; 1.8071x over previous
import functools

import jax
import jax.numpy as jnp
from jax.experimental import pallas as pl
from jax.experimental.pallas import tpu as pltpu

N = 16384
BLK = 256
NBLK = N // BLK


def _cox_kernel(x_row, t_row, e_row, t_col, e_col, out_ref):
    x = x_row[...]
    t = t_row[...]
    e = e_row[...]
    m = jnp.max(x)
    w = jnp.exp(x - m)
    term1 = jnp.sum(x * e)

    def body(b, acc):
        thr = t_col[pl.ds(b * BLK, BLK), :]
        ev = e_col[pl.ds(b * BLK, BLK), :]
        mask = t >= thr
        s = jnp.sum(jnp.where(mask, w, 0.0), axis=1, keepdims=True)
        return acc + jnp.sum(ev * (m + jnp.log(s)))

    acc = jax.lax.fori_loop(0, NBLK, body, jnp.float32(0.0))
    loss = (acc - term1) / N
    out_ref[...] = jnp.sqrt(loss)[None, None]


@jax.jit
def kernel(input, target):
    x = input.reshape(1, N)
    t = target[:, 0]
    e = target[:, 1]
    out = pl.pallas_call(
        _cox_kernel,
        out_shape=jax.ShapeDtypeStruct((1, 1), jnp.float32),
        in_specs=[
            pl.BlockSpec((1, N), lambda: (0, 0)),
            pl.BlockSpec((1, N), lambda: (0, 0)),
            pl.BlockSpec((1, N), lambda: (0, 0)),
            pl.BlockSpec((N, 1), lambda: (0, 0)),
            pl.BlockSpec((N, 1), lambda: (0, 0)),
        ],
        out_specs=pl.BlockSpec((1, 1), lambda: (0, 0)),
    )(x, t.reshape(1, N), e.reshape(1, N), t.reshape(N, 1), e.reshape(N, 1))
    return out[0, 0]
